# Initial kernel scaffold; baseline (speedup 1.0000x reference)
import jax
import jax.numpy as jnp
from jax.experimental import pallas as pl


def kernel(prop_state, indic, mask):
    raise NotImplementedError("write your pallas kernel here")


# speedup vs baseline: 7.1656x; 7.1656x over previous
import functools

import jax
import jax.numpy as jnp
from jax import lax
from jax.experimental import pallas as pl
from jax.experimental.pallas import tpu as pltpu
from jax.experimental.pallas import tpu_sc as plsc

BATCH = 4
N_NODE = 2048
STATE_DIM = 256
NNZ = 32768
EPS = 1e-8

NC = 2
NS = 16
L = 16

CHUNK_ROWS = 512
N_CHUNKS = N_NODE // CHUNK_ROWS
N_TASKS = BATCH * N_CHUNKS
TASKS_PER_CORE = N_TASKS // NC
PAIRS_PER_TILE = NNZ // NS
GROUPS_PER_TILE = PAIRS_PER_TILE // L
ACC_WORDS = CHUNK_ROWS * N_NODE
STRIPE_WORDS = ACC_WORDS // NS
DUM_WORDS = NS * 128
ZBUF_WORDS = 16384


def _sc_cnt_kernel(rows_hbm, cols_hbm, zeros_hbm, cnt_hbm, acc_sh, rows_v,
                   cols_v, addr_v, ones_v, zbuf_v):
    c = lax.axis_index("c")
    s = lax.axis_index("s")
    lane = lax.iota(jnp.int32, L)

    pltpu.sync_copy(zeros_hbm, zbuf_v)
    for g in range(128 // L):
        ones_v[pl.ds(g * L, L)] = jnp.full((L,), 1.0, jnp.float32)

    def task_body(t, carry):
        task = c + NC * t
        b = task // N_CHUNKS
        q = task % N_CHUNKS
        r0 = q * CHUNK_ROWS

        for z in range(STRIPE_WORDS // ZBUF_WORDS):
            pltpu.sync_copy(
                zbuf_v, acc_sh.at[pl.ds(s * STRIPE_WORDS + z * ZBUF_WORDS,
                                        ZBUF_WORDS)])
        pltpu.sync_copy(zbuf_v.at[pl.ds(0, 128)],
                        acc_sh.at[pl.ds(ACC_WORDS + s * 128, 128)])

        pltpu.sync_copy(rows_hbm.at[b, pl.ds(s * PAIRS_PER_TILE,
                                             PAIRS_PER_TILE)], rows_v)
        pltpu.sync_copy(cols_hbm.at[b, pl.ds(s * PAIRS_PER_TILE,
                                             PAIRS_PER_TILE)], cols_v)
        plsc.subcore_barrier()

        dum_base = ACC_WORDS + s * 128

        def group_body(g, carry2):
            i = rows_v[pl.ds(g * L, L)]
            j = cols_v[pl.ds(g * L, L)]
            dum = dum_base + (g % 8) * L + lane
            in1 = (i >= r0) & (i < r0 + CHUNK_ROWS)
            a1 = jnp.where(in1, (i - r0) * N_NODE + j, dum)
            in2 = (j >= r0) & (j < r0 + CHUNK_ROWS)
            a2 = jnp.where(in2, (j - r0) * N_NODE + i, dum)
            row = g // 4
            col = (g % 4) * 2 * L
            addr_v[row, pl.ds(col, L)] = a1
            addr_v[row, pl.ds(col + L, L)] = a2
            return carry2

        lax.fori_loop(0, GROUPS_PER_TILE, group_body, 0)

        def scat_body(k, carry2):
            pltpu.sync_copy(ones_v, acc_sh.at[addr_v.at[k]], add=True)
            return carry2

        lax.fori_loop(0, GROUPS_PER_TILE // 4, scat_body, 0)
        plsc.subcore_barrier()

        pltpu.sync_copy(
            acc_sh.at[pl.ds(s * STRIPE_WORDS, STRIPE_WORDS)],
            cnt_hbm.at[b, pl.ds(r0 * N_NODE + s * STRIPE_WORDS,
                                STRIPE_WORDS)])
        plsc.subcore_barrier()
        return carry

    lax.fori_loop(0, TASKS_PER_CORE, task_body, 0)


def _sc_cnt(rows, cols):
    zeros = jnp.zeros((ZBUF_WORDS,), jnp.float32)
    mesh = plsc.VectorSubcoreMesh(core_axis_name="c", subcore_axis_name="s")
    f = pl.kernel(
        _sc_cnt_kernel,
        out_type=jax.ShapeDtypeStruct((BATCH, N_NODE * N_NODE), jnp.float32),
        mesh=mesh,
        scratch_types=[
            pltpu.VMEM_SHARED((ACC_WORDS + DUM_WORDS,), jnp.float32),
            pltpu.VMEM((PAIRS_PER_TILE,), jnp.int32),
            pltpu.VMEM((PAIRS_PER_TILE,), jnp.int32),
            pltpu.VMEM((GROUPS_PER_TILE // 4, 128), jnp.int32),
            pltpu.VMEM((128,), jnp.float32),
            pltpu.VMEM((ZBUF_WORDS,), jnp.float32),
        ],
    )
    return f(rows, cols, zeros)


def _norm_kernel(x_ref, o_ref):
    x = x_ref[...]
    n = jnp.sqrt(jnp.sum(x * x, axis=1, keepdims=True))
    o_ref[...] = x / jnp.maximum(n, EPS)


def _normalize(ps2):
    rows = ps2.shape[0]
    blk = 1024
    return pl.pallas_call(
        _norm_kernel,
        grid=(rows // blk,),
        in_specs=[pl.BlockSpec((blk, STATE_DIM), lambda i: (i, 0))],
        out_specs=pl.BlockSpec((blk, STATE_DIM), lambda i: (i, 0)),
        out_shape=jax.ShapeDtypeStruct(ps2.shape, jnp.float32),
    )(ps2)


def _mm_kernel(a_ref, b_ref, cnt_ref, o_ref):
    a = a_ref[0]
    bm = b_ref[0]
    s = lax.dot_general(a, bm, (((1,), (1,)), ((), ())),
                        preferred_element_type=jnp.float32,
                        precision=lax.Precision.HIGHEST)
    sig = 1.0 / (1.0 + jnp.exp(-s))
    o_ref[0] = cnt_ref[0] * sig


def _mm_sig_mul(ps_norm, cnt):
    rb = 256
    grid = (BATCH, N_NODE // rb)
    return pl.pallas_call(
        _mm_kernel,
        grid=grid,
        in_specs=[
            pl.BlockSpec((1, rb, STATE_DIM), lambda b, i: (b, i, 0)),
            pl.BlockSpec((1, N_NODE, STATE_DIM), lambda b, i: (b, 0, 0)),
            pl.BlockSpec((1, rb, N_NODE), lambda b, i: (b, i, 0)),
        ],
        out_specs=pl.BlockSpec((1, rb, N_NODE), lambda b, i: (b, i, 0)),
        out_shape=jax.ShapeDtypeStruct((BATCH, N_NODE, N_NODE), jnp.float32),
    )(ps_norm, ps_norm, cnt)


def kernel(prop_state, indic, mask):
    del indic
    rows = mask[:, :, 0].astype(jnp.int32)
    cols = mask[:, :, 1].astype(jnp.int32)
    ps2 = prop_state.reshape(BATCH * N_NODE, STATE_DIM)
    ps_norm = _normalize(ps2).reshape(BATCH, N_NODE, STATE_DIM)
    cnt = _sc_cnt(rows, cols).reshape(BATCH, N_NODE, N_NODE)
    return _mm_sig_mul(ps_norm, cnt)
